# Initial kernel scaffold; baseline (speedup 1.0000x reference)
import jax
import jax.numpy as jnp
from jax.experimental import pallas as pl


def kernel(adj, inputs, W0, W1):
    raise NotImplementedError("write your pallas kernel here")


# speedup vs baseline: 1.2725x; 1.2725x over previous
import functools

import jax
import jax.numpy as jnp
from jax.experimental import pallas as pl


def _sage_layer_body(apply_relu, adj_ref, xblk_ref, feats_ref, wa_ref, wb_ref,
                     out_ref):
    a = adj_ref[...]
    p = jnp.dot(a, feats_ref[...], preferred_element_type=jnp.float32)
    deg = jnp.sum(a, axis=1, keepdims=True) + 1.0
    neigh = p / deg
    out = (jnp.dot(xblk_ref[...], wa_ref[...], preferred_element_type=jnp.float32)
           + jnp.dot(neigh, wb_ref[...], preferred_element_type=jnp.float32))
    if apply_relu:
        out = jnp.maximum(out, 0.0)
    out_ref[...] = out


def _sage_layer(adj, feats, wa, wb, apply_relu, bm):
    n, d = feats.shape
    dh = wa.shape[1]
    return pl.pallas_call(
        functools.partial(_sage_layer_body, apply_relu),
        grid=(n // bm,),
        in_specs=[
            pl.BlockSpec((bm, n), lambda i: (i, 0)),
            pl.BlockSpec((bm, d), lambda i: (i, 0)),
            pl.BlockSpec((n, d), lambda i: (0, 0)),
            pl.BlockSpec((d, dh), lambda i: (0, 0)),
            pl.BlockSpec((d, dh), lambda i: (0, 0)),
        ],
        out_specs=pl.BlockSpec((bm, dh), lambda i: (i, 0)),
        out_shape=jax.ShapeDtypeStruct((n, dh), jnp.float32),
    )(adj, feats, feats, wa, wb)


def kernel(adj, inputs, W0, W1):
    d_in = inputs.shape[1]
    wa0, wb0 = W0[:, :d_in].T, W0[:, d_in:].T
    h = _sage_layer(adj, inputs, wa0, wb0, apply_relu=True, bm=512)
    dh = h.shape[1]
    wa1, wb1 = W1[:, :dh].T, W1[:, dh:].T
    return _sage_layer(adj, h, wa1, wb1, apply_relu=False, bm=512)


# speedup vs baseline: 1.2899x; 1.0136x over previous
import functools

import jax
import jax.numpy as jnp
from jax.experimental import pallas as pl


def _sage_layer_body(apply_relu, adj_ref, xblk_ref, feats_ref, wa_ref, wb_ref,
                     out_ref):
    a = adj_ref[...]
    p = jnp.dot(a.astype(jnp.bfloat16), feats_ref[...].astype(jnp.bfloat16),
                preferred_element_type=jnp.float32)
    deg = jnp.sum(a, axis=1, keepdims=True) + 1.0
    neigh = p / deg
    out = (jnp.dot(xblk_ref[...], wa_ref[...], preferred_element_type=jnp.float32)
           + jnp.dot(neigh, wb_ref[...], preferred_element_type=jnp.float32))
    if apply_relu:
        out = jnp.maximum(out, 0.0)
    out_ref[...] = out


def _sage_layer(adj, feats, wa, wb, apply_relu, bm):
    n, d = feats.shape
    dh = wa.shape[1]
    return pl.pallas_call(
        functools.partial(_sage_layer_body, apply_relu),
        grid=(n // bm,),
        in_specs=[
            pl.BlockSpec((bm, n), lambda i: (i, 0)),
            pl.BlockSpec((bm, d), lambda i: (i, 0)),
            pl.BlockSpec((n, d), lambda i: (0, 0)),
            pl.BlockSpec((d, dh), lambda i: (0, 0)),
            pl.BlockSpec((d, dh), lambda i: (0, 0)),
        ],
        out_specs=pl.BlockSpec((bm, dh), lambda i: (i, 0)),
        out_shape=jax.ShapeDtypeStruct((n, dh), jnp.float32),
    )(adj, feats, feats, wa, wb)


def kernel(adj, inputs, W0, W1):
    d_in = inputs.shape[1]
    wa0, wb0 = W0[:, :d_in].T, W0[:, d_in:].T
    h = _sage_layer(adj, inputs, wa0, wb0, apply_relu=True, bm=512)
    dh = h.shape[1]
    wa1, wb1 = W1[:, :dh].T, W1[:, dh:].T
    return _sage_layer(adj, h, wa1, wb1, apply_relu=False, bm=512)
